# Initial kernel scaffold; baseline (speedup 1.0000x reference)
import jax
import jax.numpy as jnp
from jax.experimental import pallas as pl


def kernel(input, token_table, pos_table):
    raise NotImplementedError("write your pallas kernel here")


# speedup vs baseline: 2.3673x; 2.3673x over previous
import functools

import jax
import jax.numpy as jnp
from jax import lax
from jax.experimental import pallas as pl
from jax.experimental.pallas import tpu as pltpu
from jax.experimental.pallas import tpu_sc as plsc

NC = 2
NS = 16
NW = NC * NS
LANES = 16


@functools.partial(jax.jit, static_argnums=(3, 4, 5))
def _sc_embed(inp2, token_table, pos_table, b, l, h):
    rb = b // NW
    half = l // 2
    hc = h // LANES

    mesh = plsc.VectorSubcoreMesh(core_axis_name="c", subcore_axis_name="s")

    def body(inp_hbm, tok_hbm, pos_hbm, out_hbm, idx_v, rows_v, pos_v, sem):
        wid = lax.axis_index("s") * NC + lax.axis_index("c")
        pltpu.sync_copy(pos_hbm, pos_v)

        def row_body(i, _):
            bi = wid * rb + i
            pltpu.sync_copy(inp_hbm.at[bi], idx_v)
            cp0 = pltpu.async_copy(
                tok_hbm.at[idx_v.at[0]], rows_v.at[pl.ds(0, half)], sem)
            cp1 = pltpu.async_copy(
                tok_hbm.at[idx_v.at[1]], rows_v.at[pl.ds(half, half)], sem)
            cp0.wait()
            cp1.wait()

            def add_body(r, _):
                for c in range(hc):
                    sl = pl.ds(c * LANES, LANES)
                    rows_v[r, sl] = rows_v[r, sl] + pos_v[r, sl]
                return ()

            lax.fori_loop(0, l, add_body, ())
            pltpu.sync_copy(rows_v, out_hbm.at[bi])
            return ()

        lax.fori_loop(0, rb, row_body, ())

    call = pl.kernel(
        body,
        out_type=jax.ShapeDtypeStruct((b, l, h), jnp.float32),
        mesh=mesh,
        scratch_types=[
            pltpu.VMEM((2, half), jnp.int32),
            pltpu.VMEM((l, h), jnp.float32),
            pltpu.VMEM((l, h), jnp.float32),
            pltpu.SemaphoreType.DMA,
        ],
        compiler_params=pltpu.CompilerParams(use_tc_tiling_on_sc=False),
    )
    return call(inp2, token_table, pos_table)


def kernel(input, token_table, pos_table):
    b, l = input.shape
    h = token_table.shape[1]
    inp2 = input.reshape(b, 2, l // 2)
    pos_block = lax.slice(pos_table, (1, 0), (1 + l, h))
    return _sc_embed(inp2, token_table, pos_block, b, l, h)
